# Initial kernel scaffold; baseline (speedup 1.0000x reference)
import jax
import jax.numpy as jnp
from jax.experimental import pallas as pl


def kernel(x, edge_index, edge_attr, batch, W_self1, W_msg1, W_edge1, b1, W_self2, W_msg2, W_edge2, b2, Wv, bv, Wp, bp):
    raise NotImplementedError("write your pallas kernel here")


# speedup vs baseline: 3.9460x; 3.9460x over previous
import functools

import jax
import jax.numpy as jnp
from jax import lax
from jax.experimental import pallas as pl
from jax.experimental.pallas import tpu as pltpu
from jax.experimental.pallas import tpu_sc as plsc

N = 10000
E = 320000
D = 128
DE = 16
H = 128
A = 1024
G = 64

NC = 2
NS = 16
NW = NC * NS
C = 128
CH = -(-E // (NW * C))
E_PAD = NW * C * CH
NPT = 632
NP = NPT * NS

_f32 = jnp.float32


@functools.cache
def _mesh():
  return plsc.VectorSubcoreMesh(
      core_axis_name="c", subcore_axis_name="s", num_cores=NC, num_subcores=NS)


def _feat_scatter_body(x_hbm, srci_hbm, dsti_hbm, zf_hbm, out_hbm,
                       src_v, dst_v, rows_v, acc, sem):
  cc = lax.axis_index("c")
  tid = lax.axis_index("s")
  pltpu.sync_copy(zf_hbm, acc.at[pl.ds(tid * NPT, NPT)])
  wid = cc * NS + tid
  pltpu.sync_copy(srci_hbm.at[wid], src_v)
  pltpu.sync_copy(dsti_hbm.at[wid], dst_v)
  plsc.subcore_barrier()

  @pl.loop(0, CH)
  def _chunk(j):
    pltpu.async_copy(x_hbm.at[src_v.at[j]], rows_v, sem).wait()
    pltpu.sync_copy(rows_v, acc.at[dst_v.at[j]], add=True)

  plsc.subcore_barrier()
  pltpu.sync_copy(acc.at[pl.ds(tid * NPT, NPT)],
                  out_hbm.at[cc, pl.ds(tid * NPT, NPT)])


def _make_feat_scatter():
  return pl.kernel(
      _feat_scatter_body,
      out_type=jax.ShapeDtypeStruct((NC, NP, H), _f32),
      mesh=_mesh(),
      scratch_types=[
          pltpu.VMEM((CH, C), jnp.int32),
          pltpu.VMEM((CH, C), jnp.int32),
          pltpu.VMEM((C, H), _f32),
          pltpu.VMEM_SHARED((NP, H), _f32),
          pltpu.SemaphoreType.DMA,
      ],
  )


def _update_block(x_ref, s_ref, ea_ref, ws_ref, wm_ref, we_ref, b_ref, o_ref):
  s = s_ref[0] + s_ref[1]
  ea = ea_ref[0] + ea_ref[1]
  inv = 1.0 / jnp.maximum(ea[:, DE:DE + 1], 1.0)
  agg = (jnp.dot(s, wm_ref[...], preferred_element_type=_f32)
         + jnp.dot(ea, we_ref[...], preferred_element_type=_f32)) * inv
  o_ref[...] = jnp.maximum(
      jnp.dot(x_ref[...], ws_ref[...], preferred_element_type=_f32)
      + agg + b_ref[...], 0.0)


_RB = 1000
_NB = N // _RB


def _layer_update(x, s_parts, ea_parts, w_self, w_msg, w_edge_p, b):
  return pl.pallas_call(
      _update_block,
      grid=(_NB,),
      in_specs=[
          pl.BlockSpec((_RB, H), lambda i: (i, 0)),
          pl.BlockSpec((NC, _RB, H), lambda i: (0, i, 0)),
          pl.BlockSpec((NC, _RB, H), lambda i: (0, i, 0)),
          pl.BlockSpec((H, H), lambda i: (0, 0)),
          pl.BlockSpec((H, H), lambda i: (0, 0)),
          pl.BlockSpec((H, H), lambda i: (0, 0)),
          pl.BlockSpec((1, H), lambda i: (0, 0)),
      ],
      out_specs=pl.BlockSpec((_RB, H), lambda i: (i, 0)),
      out_shape=jax.ShapeDtypeStruct((N, H), _f32),
  )(x, s_parts, ea_parts, w_self, w_msg, w_edge_p, b)


def _final_block(h_ref, s_ref, ea_ref, batch_ref, ws_ref, wm_ref, we_ref,
                 b_ref, wvt_ref, bv_ref, wp_ref, bp_ref,
                 v_ref, p_ref, gacc, cacc):
  i = pl.program_id(0)

  @pl.when(i == 0)
  def _init():
    gacc[...] = jnp.zeros_like(gacc)
    cacc[...] = jnp.zeros_like(cacc)

  s = s_ref[0] + s_ref[1]
  ea = ea_ref[0] + ea_ref[1]
  inv = 1.0 / jnp.maximum(ea[:, DE:DE + 1], 1.0)
  agg = (jnp.dot(s, wm_ref[...], preferred_element_type=_f32)
         + jnp.dot(ea, we_ref[...], preferred_element_type=_f32)) * inv
  h2 = jnp.maximum(
      jnp.dot(h_ref[...], ws_ref[...], preferred_element_type=_f32)
      + agg + b_ref[...], 0.0)

  bt = batch_ref[0, 0, :]
  onehot = (lax.broadcasted_iota(jnp.int32, (G, _RB), 0)
            == bt[None, :]).astype(_f32)
  gacc[...] += jnp.dot(onehot, h2, preferred_element_type=_f32)
  cacc[...] += jnp.broadcast_to(
      jnp.sum(onehot, axis=1, keepdims=True), (G, H))

  @pl.when(i == _NB - 1)
  def _heads():
    g = gacc[...] / jnp.maximum(cacc[...], 1.0)
    v_ref[...] = jnp.tanh(
        jnp.sum(g * wvt_ref[...], axis=1, keepdims=True) + bv_ref[...])
    z = jnp.dot(g, wp_ref[...], preferred_element_type=_f32) + bp_ref[...]
    z = z - jnp.max(z, axis=1, keepdims=True)
    ez = jnp.exp(z)
    p_ref[...] = ez / jnp.sum(ez, axis=1, keepdims=True)


def _final_stage(h, s_parts, ea_parts, batch3, w_self, w_msg, w_edge_p, b,
                 wv_t, bv2, wp, bp2):
  return pl.pallas_call(
      _final_block,
      grid=(_NB,),
      in_specs=[
          pl.BlockSpec((_RB, H), lambda i: (i, 0)),
          pl.BlockSpec((NC, _RB, H), lambda i: (0, i, 0)),
          pl.BlockSpec((NC, _RB, H), lambda i: (0, i, 0)),
          pl.BlockSpec((1, 1, _RB), lambda i: (i, 0, 0)),
          pl.BlockSpec((H, H), lambda i: (0, 0)),
          pl.BlockSpec((H, H), lambda i: (0, 0)),
          pl.BlockSpec((H, H), lambda i: (0, 0)),
          pl.BlockSpec((1, H), lambda i: (0, 0)),
          pl.BlockSpec((1, H), lambda i: (0, 0)),
          pl.BlockSpec((1, 1), lambda i: (0, 0)),
          pl.BlockSpec((H, A), lambda i: (0, 0)),
          pl.BlockSpec((1, A), lambda i: (0, 0)),
      ],
      out_specs=[
          pl.BlockSpec((G, 1), lambda i: (0, 0)),
          pl.BlockSpec((G, A), lambda i: (0, 0)),
      ],
      out_shape=[
          jax.ShapeDtypeStruct((G, 1), _f32),
          jax.ShapeDtypeStruct((G, A), _f32),
      ],
      scratch_shapes=[
          pltpu.VMEM((G, H), _f32),
          pltpu.VMEM((G, H), _f32),
      ],
  )(h, s_parts, ea_parts, batch3, w_self, w_msg, w_edge_p, b,
    wv_t, bv2, wp, bp2)


def kernel(x, edge_index, edge_attr, batch, W_self1, W_msg1, W_edge1, b1,
           W_self2, W_msg2, W_edge2, b2, Wv, bv, Wp, bp):
  src = edge_index[0].astype(jnp.int32)
  dst = edge_index[1].astype(jnp.int32)
  pad = E_PAD - E
  src3 = jnp.concatenate([src, jnp.zeros((pad,), jnp.int32)]).reshape(
      NW, CH, C)
  dst3 = jnp.concatenate([dst, jnp.full((pad,), N, jnp.int32)]).reshape(
      NW, CH, C)
  ea128 = jnp.concatenate([
      edge_attr.astype(_f32),
      jnp.ones((E, 1), _f32),
      jnp.zeros((E, H - DE - 1), _f32)], axis=1)
  ea128 = jnp.concatenate([ea128, jnp.zeros((pad, H), _f32)])
  iota3 = jnp.arange(E_PAD, dtype=jnp.int32).reshape(NW, CH, C)
  zf = jnp.zeros((NPT, H), _f32)

  scat = _make_feat_scatter()
  eap = scat(ea128, iota3, dst3, zf)
  s1 = scat(x, src3, dst3, zf)
  wedge1p = jnp.concatenate(
      [W_edge1.astype(_f32), jnp.zeros((H - DE, H), _f32)], axis=0)
  wedge2p = jnp.concatenate(
      [W_edge2.astype(_f32), jnp.zeros((H - DE, H), _f32)], axis=0)
  h = _layer_update(x, s1, eap, W_self1, W_msg1, wedge1p, b1.reshape(1, H))
  s2 = scat(h, src3, dst3, zf)
  batch3 = batch.astype(jnp.int32).reshape(_NB, 1, _RB)
  v, p = _final_stage(
      h, s2, eap, batch3, W_self2, W_msg2, wedge2p, b2.reshape(1, H),
      Wv.reshape(1, H), bv.reshape(1, 1), Wp, bp.reshape(1, A))
  return (v, p)
